# baseline (speedup 1.0000x reference)
import jax
import jax.numpy as jnp
from jax import lax
from jax.experimental import pallas as pl
from jax.experimental.pallas import tpu as pltpu
from jax.experimental.pallas import tpu_sc as plsc

_B = 97
_L = 16
_NT = 8
_ROWS = _NT * _L

_E1 = 8
_E2 = 32
_W0 = 47
_B0 = 59
_W1 = 63
_B1 = 75
_W2 = 79
_B2 = 87
_XF = 96
_WK = _XF + _ROWS
_ST = _WK + _ROWS
_BUF = _ST + _ROWS


def _splat_i(v):
    return jnp.full((_L,), v, dtype=jnp.int32)


def _sc_body(buf_hbm, out_hbm, par_v, xf_v, wkf_v, stf_v, out_v, sem):
    wid = lax.axis_index("s")
    base = pl.multiple_of(wid * _L, _L)

    def spl(i):
        return plsc.load_gather(par_v, [_splat_i(i)])

    copies = [
        pltpu.async_copy(buf_hbm.at[pl.ds(0, 96)], par_v, sem),
        pltpu.async_copy(buf_hbm.at[pl.ds(_XF + base, _L)], xf_v, sem),
        pltpu.async_copy(buf_hbm.at[pl.ds(_WK + base, _L)], wkf_v, sem),
        pltpu.async_copy(buf_hbm.at[pl.ds(_ST + base, _L)], stf_v, sem),
    ]
    for cp in copies:
        cp.wait()

    iota = lax.iota(jnp.int32, _L)
    xf = xf_v[...]
    m = iota < (_B - base)
    zero = _splat_i(0)
    wk = jnp.where(m, plsc.bitcast(wkf_v[...], jnp.int32), zero)
    st = jnp.where(m, plsc.bitcast(stf_v[...], jnp.int32), zero)

    wk3 = wk * 3 + _splat_i(_E1)
    st3 = st * 3 + _splat_i(_E2)
    g1 = [plsc.load_gather(par_v, [wk3 + _splat_i(d)]) for d in range(3)]
    g2 = [plsc.load_gather(par_v, [st3 + _splat_i(d)]) for d in range(3)]

    row16 = iota * _L
    for j in range(8):
        o = xf * spl(_W2 + j) + spl(_B2 + j)
        plsc.store_scatter(out_v, [row16 + _splat_i(j)], o)
    for j in range(4):
        o = (g2[0] * spl(_W1 + j) + g2[1] * spl(_W1 + 4 + j)
             + g2[2] * spl(_W1 + 8 + j) + spl(_B1 + j))
        plsc.store_scatter(out_v, [row16 + _splat_i(8 + j)], o)
    for j in range(4):
        o = (g1[0] * spl(_W0 + j) + g1[1] * spl(_W0 + 4 + j)
             + g1[2] * spl(_W0 + 8 + j) + spl(_B0 + j))
        plsc.store_scatter(out_v, [row16 + _splat_i(12 + j)], o)

    pltpu.sync_copy(out_v, out_hbm.at[pl.ds(base * _L, _L * _L)])


@jax.jit
def _run(xf, wk, st, e1, e2, w0, b0, w1, b1, w2, b2):
    zpad8 = jnp.zeros((8,), jnp.float32)
    pad = jnp.zeros((_ROWS - _B,), jnp.float32)
    buf = jnp.concatenate([
        zpad8, e1.reshape(24), e2.reshape(15), w0.reshape(12), b0,
        w1.reshape(12), b1, w2.reshape(8), b2, jnp.zeros((1,), jnp.float32),
        xf, pad,
        lax.bitcast_convert_type(wk, jnp.float32), pad,
        lax.bitcast_convert_type(st, jnp.float32), pad,
    ])
    mesh = plsc.VectorSubcoreMesh(core_axis_name="c", subcore_axis_name="s",
                                  num_cores=1, num_subcores=_NT)
    f = pl.kernel(
        _sc_body,
        out_type=jax.ShapeDtypeStruct((_ROWS * _L,), jnp.float32),
        scratch_types=[
            pltpu.VMEM((96,), jnp.float32),
            pltpu.VMEM((_L,), jnp.float32),
            pltpu.VMEM((_L,), jnp.float32),
            pltpu.VMEM((_L,), jnp.float32),
            pltpu.VMEM((_L * _L,), jnp.float32),
            pltpu.SemaphoreType.DMA,
        ],
        mesh=mesh,
        compiler_params=pltpu.CompilerParams(
            needs_layout_passes=False,
            disable_bounds_checks=True,
            disable_semaphore_checks=True,
            skip_device_barrier=True,
        ),
    )
    return f(buf).reshape(_ROWS, _L)[:_B]


def kernel(X_feature, X_week, X_stamp, emb1, emb2, W0, b0, W1, b1, W2, b2):
    return _run(
        X_feature.astype(jnp.float32),
        X_week.astype(jnp.int32),
        X_stamp.astype(jnp.int32),
        emb1.astype(jnp.float32),
        emb2.astype(jnp.float32),
        W0.astype(jnp.float32),
        b0.astype(jnp.float32),
        W1.astype(jnp.float32),
        b1.astype(jnp.float32),
        W2.astype(jnp.float32),
        b2.astype(jnp.float32),
    )


# speedup vs baseline: 1.0092x; 1.0092x over previous
import jax
import jax.numpy as jnp
from jax import lax
from jax.experimental import pallas as pl
from jax.experimental.pallas import tpu as pltpu
from jax.experimental.pallas import tpu_sc as plsc

_B = 97
_L = 16
_NFULL = _B // _L
_TAIL = _B - _L * _NFULL


def _splat_i(v):
    return jnp.full((_L,), v, dtype=jnp.int32)


def _sc_body(xf_hbm, wk_hbm, st_hbm, e1_hbm, e2_hbm, w0_hbm, b0_hbm,
             w1_hbm, b1_hbm, w2_hbm, b2_hbm, out_hbm,
             xf_v, wk_v, st_v, e1_v, e2_v, w0_v, b0_v, w1_v, b1_v,
             w2_v, b2_v, out_v, sem):
    wid = lax.axis_index("s")

    def spl2(ref, r, c):
        return plsc.load_gather(ref, [_splat_i(r), _splat_i(c)])

    def spl1(ref, i):
        return plsc.load_gather(ref, [_splat_i(i)])

    def chunk(base, n):
        copies = [
            pltpu.async_copy(e1_hbm, e1_v, sem),
            pltpu.async_copy(e2_hbm, e2_v, sem),
            pltpu.async_copy(w0_hbm, w0_v.at[pl.ds(2, 3), :], sem),
            pltpu.async_copy(b0_hbm, b0_v.at[pl.ds(8, 4)], sem),
            pltpu.async_copy(w1_hbm, w1_v.at[pl.ds(2, 3), :], sem),
            pltpu.async_copy(b1_hbm, b1_v.at[pl.ds(8, 4)], sem),
            pltpu.async_copy(w2_hbm, w2_v.at[pl.ds(1, 1), :], sem),
            pltpu.async_copy(b2_hbm, b2_v.at[pl.ds(8, 8)], sem),
        ]
        if n != _L:
            xf_v[...] = jnp.zeros((_L,), jnp.float32)
            wk_v[...] = jnp.zeros((_L,), jnp.int32)
            st_v[...] = jnp.zeros((_L,), jnp.int32)
        copies += [
            pltpu.async_copy(xf_hbm.at[pl.ds(base, n)], xf_v.at[pl.ds(0, n)], sem),
            pltpu.async_copy(wk_hbm.at[pl.ds(base, n)], wk_v.at[pl.ds(0, n)], sem),
            pltpu.async_copy(st_hbm.at[pl.ds(base, n)], st_v.at[pl.ds(0, n)], sem),
        ]
        for cp in copies:
            cp.wait()

        iota = lax.iota(jnp.int32, _L)
        xf = xf_v[...]
        wk = wk_v[...]
        st = st_v[...]
        g1 = [plsc.load_gather(e1_v, [wk, _splat_i(d)]) for d in range(3)]
        g2 = [plsc.load_gather(e2_v, [st, _splat_i(d)]) for d in range(3)]

        for j in range(8):
            o = xf * spl2(w2_v, 1, j) + spl1(b2_v, 8 + j)
            plsc.store_scatter(out_v, [iota, _splat_i(j)], o)
        for j in range(4):
            o = (g2[0] * spl2(w1_v, 2, j) + g2[1] * spl2(w1_v, 3, j)
                 + g2[2] * spl2(w1_v, 4, j) + spl1(b1_v, 8 + j))
            plsc.store_scatter(out_v, [iota, _splat_i(8 + j)], o)
        for j in range(4):
            o = (g1[0] * spl2(w0_v, 2, j) + g1[1] * spl2(w0_v, 3, j)
                 + g1[2] * spl2(w0_v, 4, j) + spl1(b0_v, 8 + j))
            plsc.store_scatter(out_v, [iota, _splat_i(12 + j)], o)

        pltpu.sync_copy(out_v.at[pl.ds(0, n), :], out_hbm.at[pl.ds(base, n), :])

    @pl.when(wid < _NFULL)
    def _():
        chunk(pl.multiple_of(wid * _L, _L), _L)

    if _TAIL:
        @pl.when(wid == _NFULL)
        def _():
            chunk(_L * _NFULL, _TAIL)


@jax.jit
def _run(xf, wk, st, e1, e2, w0, b0, w1, b1, w2, b2):
    mesh = plsc.VectorSubcoreMesh(core_axis_name="c", subcore_axis_name="s",
                                  num_cores=1, num_subcores=8)
    f = pl.kernel(
        _sc_body,
        out_type=jax.ShapeDtypeStruct((_B, _L), jnp.float32),
        scratch_types=[
            pltpu.VMEM((_L,), jnp.float32),
            pltpu.VMEM((_L,), jnp.int32),
            pltpu.VMEM((_L,), jnp.int32),
            pltpu.VMEM((8, 3), jnp.float32),
            pltpu.VMEM((5, 3), jnp.float32),
            pltpu.VMEM((8, 4), jnp.float32),
            pltpu.VMEM((12,), jnp.float32),
            pltpu.VMEM((8, 4), jnp.float32),
            pltpu.VMEM((12,), jnp.float32),
            pltpu.VMEM((2, 8), jnp.float32),
            pltpu.VMEM((16,), jnp.float32),
            pltpu.VMEM((_L, _L), jnp.float32),
            pltpu.SemaphoreType.DMA,
        ],
        mesh=mesh,
        compiler_params=pltpu.CompilerParams(
            needs_layout_passes=False,
            disable_bounds_checks=True,
            disable_semaphore_checks=True,
            skip_device_barrier=True,
        ),
    )
    return f(xf, wk, st, e1, e2, w0, b0, w1, b1, w2, b2)


def kernel(X_feature, X_week, X_stamp, emb1, emb2, W0, b0, W1, b1, W2, b2):
    return _run(
        X_feature.astype(jnp.float32),
        X_week.astype(jnp.int32),
        X_stamp.astype(jnp.int32),
        emb1.astype(jnp.float32),
        emb2.astype(jnp.float32),
        W0.astype(jnp.float32),
        b0.astype(jnp.float32),
        W1.astype(jnp.float32),
        b1.astype(jnp.float32),
        W2.astype(jnp.float32),
        b2.astype(jnp.float32),
    )


# speedup vs baseline: 1.0113x; 1.0020x over previous
import jax
import jax.numpy as jnp
from jax import lax
from jax.experimental import pallas as pl
from jax.experimental.pallas import tpu as pltpu
from jax.experimental.pallas import tpu_sc as plsc

_B = 97
_L = 16
_NFULL = _B // _L
_TAIL = _B - _L * _NFULL


def _splat_i(v):
    return jnp.full((_L,), v, dtype=jnp.int32)


def _sc_body(xf_hbm, wk_hbm, st_hbm, e1_hbm, e2_hbm, w0_hbm, b0_hbm,
             w1_hbm, b1_hbm, w2_hbm, b2_hbm, out_hbm,
             xf_v, wk_v, st_v, e1_v, e2_v, w0_v, b0_v, w1_v, b1_v,
             w2_v, b2_v, out_v, sem):
    wid = lax.axis_index("s")

    def spl2(ref, r, c):
        return plsc.load_gather(ref, [_splat_i(r), _splat_i(c)])

    def spl1(ref, i):
        return plsc.load_gather(ref, [_splat_i(i)])

    def chunk(base, n):
        copies = [
            pltpu.async_copy(e1_hbm, e1_v, sem),
            pltpu.async_copy(e2_hbm, e2_v, sem),
            pltpu.async_copy(w0_hbm, w0_v.at[pl.ds(2, 3), :], sem),
            pltpu.async_copy(b0_hbm, b0_v.at[pl.ds(8, 4)], sem),
            pltpu.async_copy(w1_hbm, w1_v.at[pl.ds(2, 3), :], sem),
            pltpu.async_copy(b1_hbm, b1_v.at[pl.ds(8, 4)], sem),
            pltpu.async_copy(w2_hbm, w2_v.at[pl.ds(1, 1), :], sem),
            pltpu.async_copy(b2_hbm, b2_v.at[pl.ds(8, 8)], sem),
        ]
        if n != _L:
            xf_v[...] = jnp.zeros((_L,), jnp.float32)
            wk_v[...] = jnp.zeros((_L,), jnp.int32)
            st_v[...] = jnp.zeros((_L,), jnp.int32)
        copies += [
            pltpu.async_copy(xf_hbm.at[pl.ds(base, n)], xf_v.at[pl.ds(0, n)], sem),
            pltpu.async_copy(wk_hbm.at[pl.ds(base, n)], wk_v.at[pl.ds(0, n)], sem),
            pltpu.async_copy(st_hbm.at[pl.ds(base, n)], st_v.at[pl.ds(0, n)], sem),
        ]
        for cp in copies:
            cp.wait()

        iota = lax.iota(jnp.int32, _L)
        xf = xf_v[...]
        wk = wk_v[...]
        st = st_v[...]
        g1 = [plsc.load_gather(e1_v, [wk, _splat_i(d)]) for d in range(3)]
        g2 = [plsc.load_gather(e2_v, [st, _splat_i(d)]) for d in range(3)]

        for j in range(8):
            o = xf * spl2(w2_v, 1, j) + spl1(b2_v, 8 + j)
            plsc.store_scatter(out_v, [iota, _splat_i(j)], o)
        for j in range(4):
            o = (g2[0] * spl2(w1_v, 2, j) + g2[1] * spl2(w1_v, 3, j)
                 + g2[2] * spl2(w1_v, 4, j) + spl1(b1_v, 8 + j))
            plsc.store_scatter(out_v, [iota, _splat_i(8 + j)], o)
        for j in range(4):
            o = (g1[0] * spl2(w0_v, 2, j) + g1[1] * spl2(w0_v, 3, j)
                 + g1[2] * spl2(w0_v, 4, j) + spl1(b0_v, 8 + j))
            plsc.store_scatter(out_v, [iota, _splat_i(12 + j)], o)

        pltpu.sync_copy(out_v.at[pl.ds(0, n), :], out_hbm.at[pl.ds(base, n), :])

    @pl.when(wid < _NFULL)
    def _():
        chunk(pl.multiple_of(wid * _L, _L), _L)

    if _TAIL:
        @pl.when(wid == _NFULL)
        def _():
            chunk(_L * _NFULL, _TAIL)


@jax.jit
def _run(xf, wk, st, e1, e2, w0, b0, w1, b1, w2, b2):
    mesh = plsc.VectorSubcoreMesh(core_axis_name="c", subcore_axis_name="s",
                                  num_cores=1, num_subcores=8)
    f = pl.kernel(
        _sc_body,
        out_type=jax.ShapeDtypeStruct((_B, _L), jnp.float32),
        scratch_types=[
            pltpu.VMEM((_L,), jnp.float32),
            pltpu.VMEM((_L,), jnp.int32),
            pltpu.VMEM((_L,), jnp.int32),
            pltpu.VMEM((8, 3), jnp.float32),
            pltpu.VMEM((5, 3), jnp.float32),
            pltpu.VMEM((8, 4), jnp.float32),
            pltpu.VMEM((12,), jnp.float32),
            pltpu.VMEM((8, 4), jnp.float32),
            pltpu.VMEM((12,), jnp.float32),
            pltpu.VMEM((2, 8), jnp.float32),
            pltpu.VMEM((16,), jnp.float32),
            pltpu.VMEM((_L, _L), jnp.float32),
            pltpu.SemaphoreType.DMA,
        ],
        mesh=mesh,
        compiler_params=pltpu.CompilerParams(
            needs_layout_passes=False,
            use_tc_tiling_on_sc=False,
            disable_bounds_checks=True,
            disable_semaphore_checks=True,
            skip_device_barrier=True,
        ),
    )
    return f(xf, wk, st, e1, e2, w0, b0, w1, b1, w2, b2)


def kernel(X_feature, X_week, X_stamp, emb1, emb2, W0, b0, W1, b1, W2, b2):
    return _run(
        X_feature.astype(jnp.float32),
        X_week.astype(jnp.int32),
        X_stamp.astype(jnp.int32),
        emb1.astype(jnp.float32),
        emb2.astype(jnp.float32),
        W0.astype(jnp.float32),
        b0.astype(jnp.float32),
        W1.astype(jnp.float32),
        b1.astype(jnp.float32),
        W2.astype(jnp.float32),
        b2.astype(jnp.float32),
    )


# speedup vs baseline: 1.0353x; 1.0237x over previous
import jax
import jax.numpy as jnp
from jax import lax
from jax.experimental import pallas as pl
from jax.experimental.pallas import tpu as pltpu
from jax.experimental.pallas import tpu_sc as plsc

_B = 97
_L = 16
_NFULL = _B // _L
_TAIL = _B - _L * _NFULL


def _splat_i(v):
    return jnp.full((_L,), v, dtype=jnp.int32)


def _sc_body(xf_hbm, wk_hbm, st_hbm, e1_hbm, e2_hbm, w0_hbm, b0_hbm,
             w1_hbm, b1_hbm, w2_hbm, b2_hbm, out_hbm,
             xf_v, wk_v, st_v, e1_v, e2_v, w0_v, b0_v, w1_v, b1_v,
             w2_v, b2_v, out_v, sem):
    wid = lax.axis_index("s")

    def spl2(ref, r, c):
        return plsc.load_gather(ref, [_splat_i(r), _splat_i(c)])

    def spl1(ref, i):
        return plsc.load_gather(ref, [_splat_i(i)])

    def chunk(base, n):
        copies = [
            pltpu.async_copy(e1_hbm, e1_v, sem),
            pltpu.async_copy(e2_hbm, e2_v, sem),
            pltpu.async_copy(w0_hbm, w0_v.at[pl.ds(2, 3), :], sem),
            pltpu.async_copy(b0_hbm, b0_v.at[pl.ds(8, 4)], sem),
            pltpu.async_copy(w1_hbm, w1_v.at[pl.ds(2, 3), :], sem),
            pltpu.async_copy(b1_hbm, b1_v.at[pl.ds(8, 4)], sem),
            pltpu.async_copy(w2_hbm, w2_v.at[pl.ds(1, 1), :], sem),
            pltpu.async_copy(b2_hbm, b2_v.at[pl.ds(8, 8)], sem),
        ]
        if n != _L:
            xf_v[...] = jnp.zeros((_L,), jnp.float32)
            wk_v[...] = jnp.zeros((_L,), jnp.int32)
            st_v[...] = jnp.zeros((_L,), jnp.int32)
        copies += [
            pltpu.async_copy(xf_hbm.at[pl.ds(base, n)], xf_v.at[pl.ds(0, n)], sem),
            pltpu.async_copy(wk_hbm.at[pl.ds(base, n)], wk_v.at[pl.ds(0, n)], sem),
            pltpu.async_copy(st_hbm.at[pl.ds(base, n)], st_v.at[pl.ds(0, n)], sem),
        ]
        for cp in copies:
            cp.wait()

        iota = lax.iota(jnp.int32, _L)
        xf = xf_v[...]
        wk = wk_v[...]
        st = st_v[...]
        g1 = [plsc.load_gather(e1_v, [wk, _splat_i(d)]) for d in range(3)]
        g2 = [plsc.load_gather(e2_v, [st, _splat_i(d)]) for d in range(3)]

        for j in range(8):
            o = xf * spl2(w2_v, 1, j) + spl1(b2_v, 8 + j)
            plsc.store_scatter(out_v, [iota, _splat_i(j)], o)
        for j in range(4):
            o = (g2[0] * spl2(w1_v, 2, j) + g2[1] * spl2(w1_v, 3, j)
                 + g2[2] * spl2(w1_v, 4, j) + spl1(b1_v, 8 + j))
            plsc.store_scatter(out_v, [iota, _splat_i(8 + j)], o)
        for j in range(4):
            o = (g1[0] * spl2(w0_v, 2, j) + g1[1] * spl2(w0_v, 3, j)
                 + g1[2] * spl2(w0_v, 4, j) + spl1(b0_v, 8 + j))
            plsc.store_scatter(out_v, [iota, _splat_i(12 + j)], o)

        pltpu.sync_copy(out_v.at[pl.ds(0, n), :], out_hbm.at[pl.ds(base, n), :])

    @pl.when(wid < _NFULL)
    def _():
        chunk(pl.multiple_of(wid * _L, _L), _L)

    if _TAIL:
        @pl.when(wid == _NFULL)
        def _():
            chunk(_L * _NFULL, _TAIL)


@jax.jit
def _run(xf, wk, st, e1, e2, w0, b0, w1, b1, w2, b2):
    mesh = plsc.VectorSubcoreMesh(core_axis_name="c", subcore_axis_name="s",
                                  num_cores=1, num_subcores=8)
    f = pl.kernel(
        _sc_body,
        out_type=jax.ShapeDtypeStruct((_B, _L), jnp.float32),
        scratch_types=[
            pltpu.VMEM((_L,), jnp.float32),
            pltpu.VMEM((_L,), jnp.int32),
            pltpu.VMEM((_L,), jnp.int32),
            pltpu.VMEM((8, 3), jnp.float32),
            pltpu.VMEM((5, 3), jnp.float32),
            pltpu.VMEM((8, 4), jnp.float32),
            pltpu.VMEM((12,), jnp.float32),
            pltpu.VMEM((8, 4), jnp.float32),
            pltpu.VMEM((12,), jnp.float32),
            pltpu.VMEM((2, 8), jnp.float32),
            pltpu.VMEM((16,), jnp.float32),
            pltpu.VMEM((_L, _L), jnp.float32),
            pltpu.SemaphoreType.DMA,
        ],
        mesh=mesh,
        compiler_params=pltpu.CompilerParams(
            needs_layout_passes=False,
            disable_bounds_checks=True,
            disable_semaphore_checks=True,
            skip_device_barrier=True,
        ),
    )
    return f(xf, wk, st, e1, e2, w0, b0, w1, b1, w2, b2)


def kernel(X_feature, X_week, X_stamp, emb1, emb2, W0, b0, W1, b1, W2, b2):
    return _run(
        X_feature.astype(jnp.float32),
        X_week.astype(jnp.int32),
        X_stamp.astype(jnp.int32),
        emb1.astype(jnp.float32),
        emb2.astype(jnp.float32),
        W0.astype(jnp.float32),
        b0.astype(jnp.float32),
        W1.astype(jnp.float32),
        b1.astype(jnp.float32),
        W2.astype(jnp.float32),
        b2.astype(jnp.float32),
    )
